# baseline (speedup 1.0000x reference)
import functools
import math

import jax
import jax.numpy as jnp
from jax import lax
from jax.experimental import pallas as pl
from jax.experimental.pallas import tpu as pltpu
from jax.experimental.pallas import tpu_sc as plsc

DIM = 64
_SCALE = math.sqrt(DIM)

NC = 2
NS = 16
NW = NC * NS

CHUNK = 512


@functools.partial(jax.jit, static_argnums=(2,))
def _embed_lookup(table, idx, b_per_w):
    nchunks = b_per_w // CHUNK
    mesh = plsc.VectorSubcoreMesh(core_axis_name="c", subcore_axis_name="s")

    @functools.partial(
        pl.kernel,
        mesh=mesh,
        out_type=jax.ShapeDtypeStruct((idx.shape[0], DIM), jnp.float32),
        scratch_types=[
            pltpu.VMEM((CHUNK,), jnp.int32),
            pltpu.VMEM((CHUNK, DIM), jnp.float32),
            pltpu.SemaphoreType.DMA,
        ],
        compiler_params=pltpu.CompilerParams(use_tc_tiling_on_sc=False),
    )
    def k(table_hbm, idx_hbm, out_hbm, idx_v, rows_v, gsem):
        wid = lax.axis_index("s") * NC + lax.axis_index("c")
        base = wid * b_per_w

        def step(g, carry):
            off = base + g * CHUNK
            pltpu.sync_copy(idx_hbm.at[pl.ds(off, CHUNK)], idx_v)
            pltpu.async_copy(table_hbm.at[idx_v], rows_v, gsem).wait()

            def scale_row(r, c):
                for d in range(DIM // 16):
                    sl = pl.ds(d * 16, 16)
                    rows_v[r, sl] = rows_v[r, sl] * _SCALE
                return c

            lax.fori_loop(0, CHUNK, scale_row, 0)
            pltpu.sync_copy(rows_v, out_hbm.at[pl.ds(off, CHUNK)])
            return carry

        lax.fori_loop(0, nchunks, step, 0)

    return k(table, idx)


def kernel(input_ids, embedding):
    b = input_ids.size
    flat = input_ids.reshape(-1).astype(jnp.int32)
    out = _embed_lookup(embedding, flat, b // NW)
    return out.reshape(*input_ids.shape, DIM)


# speedup vs baseline: 1.1368x; 1.1368x over previous
import functools
import math

import jax
import jax.numpy as jnp
from jax import lax
from jax.experimental import pallas as pl
from jax.experimental.pallas import tpu as pltpu
from jax.experimental.pallas import tpu_sc as plsc

DIM = 64
_SCALE = math.sqrt(DIM)

NC = 2
NS = 16
NW = NC * NS

CHUNK = 256
NBUF = 4
LOOKAHEAD = 2


@functools.partial(jax.jit, static_argnums=(2,))
def _embed_lookup(table, idx, b_per_w):
    nchunks = b_per_w // CHUNK
    mesh = plsc.VectorSubcoreMesh(core_axis_name="c", subcore_axis_name="s")

    @functools.partial(
        pl.kernel,
        mesh=mesh,
        out_type=jax.ShapeDtypeStruct((idx.shape[0], DIM), jnp.float32),
        scratch_types=[
            pltpu.VMEM((b_per_w,), jnp.int32),
            pltpu.VMEM((NBUF, CHUNK, DIM), jnp.float32),
            pltpu.SemaphoreType.DMA((NBUF,)),
            pltpu.SemaphoreType.DMA((NBUF,)),
        ],
        compiler_params=pltpu.CompilerParams(use_tc_tiling_on_sc=False),
    )
    def k(table_hbm, idx_hbm, out_hbm, idx_all, rows_v, gsem, osem):
        wid = lax.axis_index("s") * NC + lax.axis_index("c")
        base = wid * b_per_w

        pltpu.sync_copy(idx_hbm.at[pl.ds(base, b_per_w)], idx_all)

        for b in range(LOOKAHEAD):
            pltpu.async_copy(
                table_hbm.at[idx_all.at[pl.ds(b * CHUNK, CHUNK)]],
                rows_v.at[b], gsem.at[b])

        @pl.loop(0, nchunks // NBUF)
        def outer(p):
            for b in range(NBUF):
                g = p * NBUF + b
                pltpu.make_async_copy(
                    table_hbm.at[pl.ds(0, CHUNK)], rows_v.at[b],
                    gsem.at[b]).wait()

                nb = (b + LOOKAHEAD) % NBUF
                gn = g + LOOKAHEAD

                @pl.when(jnp.logical_and(gn < nchunks,
                                         g >= NBUF - LOOKAHEAD))
                def _drain():
                    pltpu.make_async_copy(
                        rows_v.at[nb], out_hbm.at[pl.ds(0, CHUNK)],
                        osem.at[nb]).wait()

                @pl.when(gn < nchunks)
                def _prefetch():
                    pltpu.async_copy(
                        table_hbm.at[idx_all.at[pl.ds(gn * CHUNK, CHUNK)]],
                        rows_v.at[nb], gsem.at[nb])

                @plsc.parallel_loop(0, CHUNK, unroll=8)
                def scale(r):
                    for d in range(DIM // 16):
                        sl = pl.ds(d * 16, 16)
                        rows_v[b, r, sl] = rows_v[b, r, sl] * _SCALE

                pltpu.async_copy(
                    rows_v.at[b], out_hbm.at[pl.ds(base + g * CHUNK, CHUNK)],
                    osem.at[b])

        for b in range(NBUF):
            pltpu.make_async_copy(
                rows_v.at[b], out_hbm.at[pl.ds(0, CHUNK)], osem.at[b]).wait()

    return k(table, idx)


def kernel(input_ids, embedding):
    b = input_ids.size
    flat = input_ids.reshape(-1).astype(jnp.int32)
    out = _embed_lookup(embedding, flat, b // NW)
    return out.reshape(*input_ids.shape, DIM)


# speedup vs baseline: 1.8425x; 1.6208x over previous
import functools
import math

import jax
import jax.numpy as jnp
from jax import lax
from jax.experimental import pallas as pl
from jax.experimental.pallas import tpu as pltpu
from jax.experimental.pallas import tpu_sc as plsc

DIM = 64
_SCALE = math.sqrt(DIM)

NC = 2
NS = 16
NW = NC * NS

GROUP = 128
NROW = 4
LOOKAHEAD = 2
TP = 129


@jax.jit
def _embed_lookup(table, idx2d):
    ngroups = idx2d.shape[0]
    g_per_w = ngroups // NW
    sdim = 200
    btiles = 32
    dtiles = DIM // 8
    mesh = plsc.VectorSubcoreMesh(core_axis_name="c", subcore_axis_name="s")

    @functools.partial(
        pl.kernel,
        mesh=mesh,
        out_type=jax.ShapeDtypeStruct((sdim, dtiles, btiles, 8, GROUP),
                                      jnp.float32),
        scratch_types=[
            pltpu.VMEM((g_per_w, GROUP), jnp.int32),
            pltpu.VMEM((NROW, GROUP, DIM), jnp.float32),
            pltpu.VMEM((2, DIM, TP), jnp.float32),
            pltpu.SemaphoreType.DMA((NROW,)),
            pltpu.SemaphoreType.DMA((2,)),
        ],
        compiler_params=pltpu.CompilerParams(use_tc_tiling_on_sc=False,
                                             needs_layout_passes=False),
    )
    def k(table_hbm, idx_hbm, out_hbm, idx_all, rows_v, trans_v, gsem, osem):
        wid = lax.axis_index("s") * NC + lax.axis_index("c")
        r0 = wid * g_per_w

        pltpu.sync_copy(idx_hbm.at[pl.ds(r0, g_per_w)], idx_all)

        for q in range(LOOKAHEAD):
            pltpu.async_copy(table_hbm.at[idx_all.at[q]], rows_v.at[q],
                             gsem.at[q])

        lane = jnp.arange(16, dtype=jnp.int32)

        @pl.loop(0, g_per_w // NROW)
        def outer(p):
            for q in range(NROW):
                g = p * NROW + q
                tb = q % 2

                pltpu.make_async_copy(
                    table_hbm.at[pl.ds(0, GROUP)], rows_v.at[q],
                    gsem.at[q]).wait()

                nq = (q + LOOKAHEAD) % NROW
                gn = g + LOOKAHEAD

                @pl.when(gn < g_per_w)
                def _prefetch():
                    pltpu.async_copy(
                        table_hbm.at[idx_all.at[gn]], rows_v.at[nq],
                        gsem.at[nq])

                r = r0 + g
                ss = lax.rem(r, 8)
                bt = lax.rem(lax.div(r, 8), btiles)
                s = lax.div(r, 8 * btiles) * 8 + ss

                @pl.when(g >= 2)
                def _drain():
                    pltpu.make_async_copy(
                        trans_v.at[tb, :, pl.ds(0, GROUP)],
                        out_hbm.at[0, :, 0], osem.at[tb]).wait()

                @plsc.parallel_loop(0, GROUP, unroll=4)
                def transpose(t):
                    col = jnp.full((16,), t, dtype=jnp.int32)
                    for c in range(DIM // 16):
                        v = rows_v[q, t, pl.ds(c * 16, 16)] * _SCALE
                        plsc.store_scatter(
                            trans_v.at[tb], [c * 16 + lane, col], v)

                for dt in range(dtiles):
                    pltpu.async_copy(
                        trans_v.at[tb, pl.ds(dt * 8, 8), pl.ds(0, GROUP)],
                        out_hbm.at[s, dt, bt], osem.at[tb])

        for tb in range(2):
            pltpu.make_async_copy(
                trans_v.at[tb, :, pl.ds(0, GROUP)],
                out_hbm.at[0, :, 0], osem.at[tb]).wait()

    return k(table, idx2d)


def kernel(input_ids, embedding):
    nb, ns = input_ids.shape
    idx2d = (input_ids.T.reshape(ns // 8, 8, nb // 128, 128)
             .transpose(0, 2, 1, 3)
             .reshape(ns // 8 * (nb // 128) * 8, 128)
             .astype(jnp.int32))
    out5d = _embed_lookup(embedding, idx2d)
    return out5d.transpose(2, 4, 0, 1, 3).reshape(nb, ns, DIM)
